# Initial kernel scaffold; baseline (speedup 1.0000x reference)
import jax
import jax.numpy as jnp
from jax.experimental import pallas as pl


def kernel(x, edge_index, W0, W1):
    raise NotImplementedError("write your pallas kernel here")


# speedup vs baseline: 11.0497x; 11.0497x over previous
import functools

import jax
import jax.numpy as jnp
from jax import lax
from jax.experimental import pallas as pl
from jax.experimental.pallas import tpu as pltpu
from jax.experimental.pallas import tpu_sc as plsc

NC = 2
NS = 16
LANES = 16


def _deg_hist(row, n_pad, chunk):
  e = row.shape[0]
  per_w = e // (NC * NS)
  n_chunks = per_w // chunk
  assert per_w % chunk == 0 and per_w % 8 == 0 and chunk % 8 == 0
  slab = n_pad // NS
  assert n_pad % NS == 0 and slab % 8 == 0

  mesh = plsc.VectorSubcoreMesh(core_axis_name="c", subcore_axis_name="s")

  @functools.partial(
      pl.kernel,
      out_type=jax.ShapeDtypeStruct((NC, n_pad), jnp.float32),
      mesh=mesh,
      scratch_types=[
          pltpu.VMEM((chunk,), jnp.int32),
          pltpu.VMEM((chunk,), jnp.float32),
          pltpu.VMEM((slab,), jnp.float32),
          pltpu.VMEM_SHARED((n_pad,), jnp.float32),
      ],
  )
  def k(row_hbm, out_hbm, ridx, ones, zbuf, acc):
    c = lax.axis_index("c")
    s = lax.axis_index("s")
    wid = c * NS + s

    @pl.loop(0, slab // LANES)
    def _(i):
      zbuf[pl.ds(i * LANES, LANES)] = jnp.zeros((LANES,), jnp.float32)

    @pl.loop(0, chunk // LANES)
    def _(i):
      ones[pl.ds(i * LANES, LANES)] = jnp.ones((LANES,), jnp.float32)

    pltpu.sync_copy(zbuf, acc.at[pl.ds(s * slab, slab)])
    plsc.subcore_barrier()

    @pl.loop(0, n_chunks)
    def _(j):
      base = wid * per_w + j * chunk
      pltpu.sync_copy(row_hbm.at[pl.ds(base, chunk)], ridx)
      pltpu.sync_copy(ones, acc.at[ridx], add=True)

    plsc.subcore_barrier()
    pltpu.sync_copy(acc.at[pl.ds(s * slab, slab)],
                    out_hbm.at[c].at[pl.ds(s * slab, slab)])

  return k(row)


def _seg_rows(g, row, col, chunk):
  n_pad, d = g.shape
  e = row.shape[0]
  per_w = e // (NC * NS)
  n_chunks = per_w // chunk
  assert per_w % chunk == 0 and per_w % 8 == 0 and chunk % 8 == 0
  slab = n_pad // NS
  assert n_pad % NS == 0 and slab % 8 == 0
  assert slab % chunk == 0

  mesh = plsc.VectorSubcoreMesh(core_axis_name="c", subcore_axis_name="s")

  @functools.partial(
      pl.kernel,
      out_type=jax.ShapeDtypeStruct((NC, n_pad, d), jnp.float32),
      mesh=mesh,
      scratch_types=[
          pltpu.VMEM((chunk,), jnp.int32),
          pltpu.VMEM((chunk,), jnp.int32),
          pltpu.VMEM((chunk, d), jnp.float32),
          pltpu.VMEM_SHARED((n_pad, d), jnp.float32),
          pltpu.SemaphoreType.DMA,
      ],
  )
  def k(g_hbm, row_hbm, col_hbm, out_hbm, cidx, ridx, rows, acc, sem):
    c = lax.axis_index("c")
    s = lax.axis_index("s")
    wid = c * NS + s

    @pl.loop(0, chunk)
    def _(i):
      @pl.loop(0, d // LANES)
      def _(j):
        rows[i, pl.ds(j * LANES, LANES)] = jnp.zeros((LANES,), jnp.float32)

    @pl.loop(0, slab // chunk)
    def _(i):
      pltpu.sync_copy(rows, acc.at[pl.ds(s * slab + i * chunk, chunk)])

    plsc.subcore_barrier()

    @pl.loop(0, n_chunks)
    def _(j):
      base = wid * per_w + j * chunk
      pltpu.sync_copy(col_hbm.at[pl.ds(base, chunk)], cidx)
      pltpu.sync_copy(row_hbm.at[pl.ds(base, chunk)], ridx)
      pltpu.async_copy(g_hbm.at[cidx], rows, sem).wait()
      pltpu.sync_copy(rows, acc.at[ridx], add=True)

    plsc.subcore_barrier()
    pltpu.sync_copy(acc.at[pl.ds(s * slab, slab)],
                    out_hbm.at[c].at[pl.ds(s * slab, slab)])

  return k(g, row, col)


def _mm_body(x_ref, w_ref, o_ref):
  o_ref[...] = jnp.dot(x_ref[...], w_ref[...],
                       preferred_element_type=jnp.float32)


def _matmul(x, w, blk):
  n, d = x.shape
  return pl.pallas_call(
      _mm_body,
      grid=(n // blk,),
      in_specs=[
          pl.BlockSpec((blk, d), lambda i: (i, 0)),
          pl.BlockSpec((d, d), lambda i: (0, 0)),
      ],
      out_specs=pl.BlockSpec((blk, d), lambda i: (i, 0)),
      out_shape=jax.ShapeDtypeStruct((n, d), jnp.float32),
  )(x, w)


def _scale_body(degp_ref, xw_ref, dinv_ref, g_ref):
  deg = 1.0 + degp_ref[0] + degp_ref[1]
  dinv = lax.rsqrt(deg)
  dinv_ref[...] = dinv
  g_ref[...] = dinv * xw_ref[...]


def _scale(degp, xw, blk):
  n, d = xw.shape
  return pl.pallas_call(
      _scale_body,
      grid=(n // blk,),
      in_specs=[
          pl.BlockSpec((NC, blk, 1), lambda i: (0, i, 0)),
          pl.BlockSpec((blk, d), lambda i: (i, 0)),
      ],
      out_specs=[
          pl.BlockSpec((blk, 1), lambda i: (i, 0)),
          pl.BlockSpec((blk, d), lambda i: (i, 0)),
      ],
      out_shape=[
          jax.ShapeDtypeStruct((n, 1), jnp.float32),
          jax.ShapeDtypeStruct((n, d), jnp.float32),
      ],
  )(degp, xw)


def _mid_body(accp_ref, g_ref, dinv_ref, w_ref, g2_ref):
  dinv = dinv_ref[...]
  h = accp_ref[0] + accp_ref[1] + g_ref[...]
  h = jnp.maximum(dinv * h, 0.0)
  g2_ref[...] = dinv * jnp.dot(h, w_ref[...],
                               preferred_element_type=jnp.float32)


def _mid(accp, g, dinv, w, blk):
  n, d = g.shape
  return pl.pallas_call(
      _mid_body,
      grid=(n // blk,),
      in_specs=[
          pl.BlockSpec((NC, blk, d), lambda i: (0, i, 0)),
          pl.BlockSpec((blk, d), lambda i: (i, 0)),
          pl.BlockSpec((blk, 1), lambda i: (i, 0)),
          pl.BlockSpec((d, d), lambda i: (0, 0)),
      ],
      out_specs=pl.BlockSpec((blk, d), lambda i: (i, 0)),
      out_shape=jax.ShapeDtypeStruct((n, d), jnp.float32),
  )(accp, g, dinv, w)


def _final_body(accp_ref, g_ref, dinv_ref, o_ref):
  dinv = dinv_ref[...]
  h = accp_ref[0] + accp_ref[1] + g_ref[...]
  h = jnp.maximum(dinv * h, 0.0)
  m = jnp.max(h, axis=-1, keepdims=True)
  ex = jnp.exp(h - m)
  o_ref[...] = ex / jnp.sum(ex, axis=-1, keepdims=True)


def _final(accp, g, dinv, blk):
  n, d = g.shape
  return pl.pallas_call(
      _final_body,
      grid=(n // blk,),
      in_specs=[
          pl.BlockSpec((NC, blk, d), lambda i: (0, i, 0)),
          pl.BlockSpec((blk, d), lambda i: (i, 0)),
          pl.BlockSpec((blk, 1), lambda i: (i, 0)),
      ],
      out_specs=pl.BlockSpec((blk, d), lambda i: (i, 0)),
      out_shape=jax.ShapeDtypeStruct((n, d), jnp.float32),
  )(accp, g, dinv)


def kernel(x, edge_index, W0, W1):
  n, d = x.shape
  e = edge_index.shape[1]
  row = edge_index[0]
  col = edge_index[1]

  n_pad = ((n + 1279) // 1280) * 1280
  x_p = jnp.pad(x, ((0, n_pad - n), (0, 0)))

  chunk = 80
  blk = 1024

  degp = _deg_hist(row, n_pad, chunk)
  xw0 = _matmul(x_p, W0, blk)
  dinv, g0 = _scale(degp.reshape(NC, n_pad, 1), xw0, blk)
  acc0 = _seg_rows(g0, row, col, chunk)
  g1 = _mid(acc0, g0, dinv, W1, blk)
  acc1 = _seg_rows(g1, row, col, chunk)
  out = _final(acc1, g1, dinv, blk)
  return out[:n]
